# baseline (speedup 1.0000x reference)
import functools

import jax
import jax.numpy as jnp
from jax import lax
from jax.experimental import pallas as pl
from jax.experimental.pallas import tpu as pltpu
from jax.experimental.pallas import tpu_sc as plsc

LANES = 16
NC, NS = 2, 16
NW = NC * NS
CHUNK = 16
SUP = 128
NQ = 4
NHALF = 2
NSLOT = NQ * NHALF


def _pre_body(seg_ref, pos_ref, pos0_ref, delta_ref):
    pos0_ref[...] = pos_ref[...] + seg_ref[0:1, :]
    delta_ref[...] = seg_ref[1:2, :] - seg_ref[0:1, :]


def _make_pre(seg, pos):
    t, d = seg.shape
    s = pos.shape[0]
    assert t == 2
    return pl.pallas_call(
        _pre_body,
        out_shape=(jax.ShapeDtypeStruct((s, d), jnp.float32),
                   jax.ShapeDtypeStruct((1, d), jnp.float32)),
    )(seg, pos)


def _sc_body(seq, total_rows, d,
             tok_hbm, pos0_hbm, delta_hbm, ids2_hbm, tt2_hbm, out_hbm,
             idlist, idsblk, ttblk, pos0blk, delta_v, bufs,
             sem_i, sem_g, sem_w):
    vecs = d // LANES
    rows_per_w = total_rows // NW
    seqs_per_w = rows_per_w // seq
    nsup = seq // SUP
    sub_blocks = SUP // CHUNK
    ngroups = seqs_per_w // NQ
    miters = ngroups // NHALF
    wid = lax.axis_index("s") * NC + lax.axis_index("c")
    wbase = wid * rows_per_w
    wrow0 = wid * (rows_per_w // SUP)
    iota = lax.iota(jnp.int32, LANES)
    zero16 = iota * 0

    pltpu.sync_copy(delta_hbm, delta_v)

    def splat(v, l):
        idx = (zero16 + l)[:, None]
        dn = lax.GatherDimensionNumbers(
            offset_dims=(), collapsed_slice_dims=(0,), start_index_map=(0,))
        return lax.gather(v, idx, dn, slice_sizes=(1,),
                          mode=lax.GatherScatterMode.PROMISE_IN_BOUNDS)

    def issue_group(half, g, p2):
        for s in range(NQ):
            q = g * NQ + s
            pltpu.async_copy(
                tok_hbm.at[idsblk.at[q, pl.ds(p2 * CHUNK, CHUNK)]],
                bufs.at[half * NQ + s], sem_g[half * NQ + s])

    def wait_group(half):
        for s in range(NQ):
            slot = half * NQ + s
            pltpu.make_async_copy(tok_hbm.at[pl.ds(0, CHUNK)],
                                  bufs.at[slot], sem_g[slot]).wait()

    def write_group(half, g, p):
        for s in range(NQ):
            q = g * NQ + s
            base = wbase + q * seq + p * CHUNK
            pltpu.async_copy(bufs.at[half * NQ + s],
                             out_hbm.at[pl.ds(base, CHUNK)],
                             sem_w[half * NQ + s])

    def drain_group(half):
        for s in range(NQ):
            slot = half * NQ + s
            pltpu.make_async_copy(bufs.at[slot],
                                  out_hbm.at[pl.ds(0, CHUNK)],
                                  sem_w[slot]).wait()

    def vadd_group(half, g, p2):
        tfs = [ttblk[g * NQ + s, pl.ds(p2 * CHUNK, CHUNK)].astype(jnp.float32)
               for s in range(NQ)]

        @pl.loop(0, CHUNK)
        def _row(i):
            tf = [splat(tfs[s], i) for s in range(NQ)]
            for j in range(vecs):
                sl = pl.ds(j * LANES, LANES)
                dv = delta_v[0, sl]
                pv = pos0blk[i, sl]
                for s in range(NQ):
                    buf = bufs.at[half * NQ + s]
                    buf[i, sl] = buf[i, sl] + (pv + tf[s] * dv)

    @pl.loop(0, nsup)
    def _sup(sup):
        for j in range(seqs_per_w // LANES):
            idlist[pl.ds(j * LANES, LANES)] = (
                wrow0 + (iota + j * LANES) * nsup + sup)
        cp1 = pltpu.async_copy(ids2_hbm.at[idlist], idsblk, sem_i)
        cp2 = pltpu.async_copy(tt2_hbm.at[idlist], ttblk, sem_i)
        cp1.wait()
        cp2.wait()

        @pl.loop(0, sub_blocks)
        def _pblock(p2):
            p = sup * sub_blocks + p2
            pltpu.sync_copy(pos0_hbm.at[pl.ds(p * CHUNK, CHUNK)], pos0blk)

            issue_group(0, 0, p2)
            issue_group(1, 1, p2)

            @pl.loop(0, miters)
            def _m(m):
                wait_group(0)
                vadd_group(0, 2 * m, p2)
                write_group(0, 2 * m, p)
                wait_group(1)
                vadd_group(1, 2 * m + 1, p2)
                write_group(1, 2 * m + 1, p)

                @pl.when(m < miters - 1)
                def _reissue():
                    drain_group(0)
                    issue_group(0, 2 * m + 2, p2)
                    drain_group(1)
                    issue_group(1, 2 * m + 3, p2)

            drain_group(0)
            drain_group(1)


@functools.lru_cache(maxsize=None)
def _make_sc(seq, total_rows, d):
    rows_per_w = total_rows // NW
    seqs_per_w = rows_per_w // seq
    assert total_rows % NW == 0 and rows_per_w % seq == 0
    assert seq % SUP == 0 and seqs_per_w % (NQ * NHALF) == 0
    assert seqs_per_w % LANES == 0 and d % LANES == 0
    mesh = plsc.VectorSubcoreMesh(
        core_axis_name="c", subcore_axis_name="s",
        num_cores=NC, num_subcores=NS)
    return pl.kernel(
        functools.partial(_sc_body, seq, total_rows, d),
        out_type=jax.ShapeDtypeStruct((total_rows, d), jnp.float32),
        mesh=mesh,
        scratch_types=[
            pltpu.VMEM((seqs_per_w,), jnp.int32),
            pltpu.VMEM((seqs_per_w, SUP), jnp.int32),
            pltpu.VMEM((seqs_per_w, SUP), jnp.int32),
            pltpu.VMEM((CHUNK, d), jnp.float32),
            pltpu.VMEM((1, d), jnp.float32),
            pltpu.VMEM((NSLOT, CHUNK, d), jnp.float32),
            pltpu.SemaphoreType.DMA,
            [pltpu.SemaphoreType.DMA] * NSLOT,
            [pltpu.SemaphoreType.DMA] * NSLOT,
        ],
    )


def kernel(input_ids, token_type_ids, token_embedding, segment_embedding,
           position_embedding):
    b, s = input_ids.shape
    d = token_embedding.shape[1]
    pos0, delta = _make_pre(segment_embedding, position_embedding)
    ids2 = input_ids.reshape(-1, SUP).astype(jnp.int32)
    tt2 = token_type_ids.reshape(-1, SUP).astype(jnp.int32)
    sc = _make_sc(s, b * s, d)
    out = sc(token_embedding, pos0, delta, ids2, tt2)
    return out.reshape(b, s, d)
